# baseline (speedup 1.0000x reference)
import jax
import jax.numpy as jnp
from jax import lax
from jax.experimental import pallas as pl
from jax.experimental.pallas import tpu as pltpu
from jax.experimental.pallas import tpu_sc as plsc

B, T, D, H = 1024, 200, 64, 128
BT = B * T
NC, NS = 2, 16
NW = NC * NS
CHUNK = 128
NCHUNKS = BT // CHUNK
CPW = NCHUNKS // NW
SPG = 4
TB = T // SPG
BC = 512
NBC = B // BC


def _sc_gather_body(table_hbm, idx_hbm, out_hbm, idx_v, rows0, rows1, sem0, sem1):
    wid = lax.axis_index("s") * NC + lax.axis_index("c")
    rbase = wid * CPW
    pltpu.sync_copy(idx_hbm.at[pl.ds(rbase * CHUNK, CPW * CHUNK)], idx_v)

    @pl.loop(0, CPW // 2)
    def _pair(p):
        c0 = p * 2
        d0 = pltpu.async_copy(
            table_hbm.at[idx_v.at[pl.ds(c0 * CHUNK, CHUNK)]], rows0, sem0)
        d1 = pltpu.async_copy(
            table_hbm.at[idx_v.at[pl.ds((c0 + 1) * CHUNK, CHUNK)]], rows1, sem1)
        d0.wait()
        pltpu.sync_copy(rows0, out_hbm.at[pl.ds((rbase + c0) * CHUNK, CHUNK)])
        d1.wait()
        pltpu.sync_copy(rows1, out_hbm.at[pl.ds((rbase + c0 + 1) * CHUNK, CHUNK)])


def _sc_gather(table, idx):
    return pl.kernel(
        _sc_gather_body,
        out_type=jax.ShapeDtypeStruct((BT, D), jnp.bfloat16),
        mesh=plsc.VectorSubcoreMesh(
            core_axis_name="c", subcore_axis_name="s", num_cores=NC, num_subcores=NS
        ),
        scratch_types=[
            pltpu.VMEM((CPW * CHUNK,), jnp.int32),
            pltpu.VMEM((CHUNK, D), jnp.bfloat16),
            pltpu.VMEM((CHUNK, D), jnp.bfloat16),
            pltpu.SemaphoreType.DMA,
            pltpu.SemaphoreType.DMA,
        ],
        compiler_params=pltpu.CompilerParams(use_tc_tiling_on_sc=False),
    )(table, idx)


def _lstm_cell(x_cat, c_prev, w_ref, b_ref):
    g = jnp.dot(x_cat, w_ref[...], preferred_element_type=jnp.float32) + b_ref[...]
    i = 0.5 * jnp.tanh(g[:, 0:H]) + 0.5
    f = 0.5 * jnp.tanh(g[:, H:2 * H]) + 0.5
    gg = jnp.tanh(g[:, 2 * H:3 * H])
    o = 0.5 * jnp.tanh(g[:, 3 * H:4 * H]) + 0.5
    c2 = f * c_prev + i * gg
    h2 = o * jnp.tanh(c2)
    return h2, c2


def _fused_body(ef_ref, eb_ref, wf_ref, wb_ref, bf_ref, bb_ref,
                w1f_ref, b1f_ref, w1b_ref, b1b_ref, fcw_ref, fcb_ref,
                out_ref, of_s, ob_s, hf, cf, hb, cb, h1, c1):
    p = pl.program_id(1)
    t = pl.program_id(2)

    @pl.when(t == 0)
    def _init():
        @pl.when(p == 0)
        def _():
            hf[...] = jnp.zeros_like(hf)
            cf[...] = jnp.zeros_like(cf)
            hb[...] = jnp.zeros_like(hb)
            cb[...] = jnp.zeros_like(cb)

        @pl.when(p == 1)
        def _():
            h1[...] = jnp.zeros_like(h1)
            c1[...] = jnp.zeros_like(c1)

    @pl.when(p == 0)
    def _layer0():
        for s in range(SPG):
            xf = jnp.concatenate([ef_ref[s], hf[...].astype(jnp.bfloat16)], axis=1)
            h2f, c2f = _lstm_cell(xf, cf[...], wf_ref, bf_ref)
            hf[...] = h2f
            cf[...] = c2f
            of_s[t * SPG + s] = h2f.astype(jnp.bfloat16)

            xb = jnp.concatenate([eb_ref[SPG - 1 - s], hb[...].astype(jnp.bfloat16)], axis=1)
            h2b, c2b = _lstm_cell(xb, cb[...], wb_ref, bb_ref)
            hb[...] = h2b
            cb[...] = c2b
            ob_s[(TB - 1 - t) * SPG + (SPG - 1 - s)] = h2b.astype(jnp.bfloat16)

    @pl.when(p == 1)
    def _layer1():
        cat = None
        h2 = None
        for s in range(SPG):
            ta = t * SPG + s
            cat = jnp.concatenate([of_s[ta], ob_s[ta]], axis=1)
            x = jnp.concatenate([cat, h1[...].astype(jnp.bfloat16)], axis=1)
            h2, c2 = _lstm_cell(x, c1[...], w1f_ref, b1f_ref)
            h1[...] = h2
            c1[...] = c2

        @pl.when(t == TB - 1)
        def _head():
            gb = jnp.dot(cat, w1b_ref[...], preferred_element_type=jnp.float32) + b1b_ref[...]
            ib = 0.5 * jnp.tanh(gb[:, 0:H]) + 0.5
            ggb = jnp.tanh(gb[:, 2 * H:3 * H])
            ob_g = 0.5 * jnp.tanh(gb[:, 3 * H:4 * H]) + 0.5
            cb2 = ib * ggb
            hb2 = ob_g * jnp.tanh(cb2)
            hcat = jnp.concatenate([h2, hb2], axis=1)
            out_ref[...] = (
                jnp.dot(hcat, fcw_ref[...], preferred_element_type=jnp.float32)
                + fcb_ref[...]
            )


def _run_fused(e_tbd, wf, wb, bf, bb, w1f, b1f, w1b, b1b, fcw, fcb):
    const = lambda bc, p, t: (0, 0)
    return pl.pallas_call(
        _fused_body,
        grid=(NBC, 2, TB),
        in_specs=[
            pl.BlockSpec((SPG, BC, D),
                         lambda bc, p, t: (jnp.where(p == 0, t, 0), bc, 0)),
            pl.BlockSpec((SPG, BC, D),
                         lambda bc, p, t: (jnp.where(p == 0, TB - 1 - t, 0), bc, 0)),
            pl.BlockSpec((D + H, 4 * H), const),
            pl.BlockSpec((D + H, 4 * H), const),
            pl.BlockSpec((1, 4 * H), const),
            pl.BlockSpec((1, 4 * H), const),
            pl.BlockSpec((3 * H, 4 * H), const),
            pl.BlockSpec((1, 4 * H), const),
            pl.BlockSpec((2 * H, 4 * H), const),
            pl.BlockSpec((1, 4 * H), const),
            pl.BlockSpec((2 * H, 1), const),
            pl.BlockSpec((1, 1), const),
        ],
        out_specs=pl.BlockSpec((BC, 1), lambda bc, p, t: (bc, 0)),
        out_shape=jax.ShapeDtypeStruct((B, 1), jnp.float32),
        scratch_shapes=[
            pltpu.VMEM((T, BC, H), jnp.bfloat16),
            pltpu.VMEM((T, BC, H), jnp.bfloat16),
            pltpu.VMEM((BC, H), jnp.float32),
            pltpu.VMEM((BC, H), jnp.float32),
            pltpu.VMEM((BC, H), jnp.float32),
            pltpu.VMEM((BC, H), jnp.float32),
            pltpu.VMEM((BC, H), jnp.float32),
            pltpu.VMEM((BC, H), jnp.float32),
        ],
        compiler_params=pltpu.CompilerParams(
            dimension_semantics=("arbitrary", "arbitrary", "arbitrary"),
            vmem_limit_bytes=120 * 1024 * 1024,
        ),
    )(e_tbd, e_tbd, wf, wb, bf, bb, w1f, b1f, w1b, b1b, fcw, fcb)


def _gate_scale():
    return jnp.concatenate([
        jnp.full((H,), 0.5, jnp.float32),
        jnp.full((H,), 0.5, jnp.float32),
        jnp.ones((H,), jnp.float32),
        jnp.full((H,), 0.5, jnp.float32),
    ])


def kernel(x, emb,
           W_ih_l0_f, W_hh_l0_f, b_ih_l0_f, b_hh_l0_f,
           W_ih_l0_b, W_hh_l0_b, b_ih_l0_b, b_hh_l0_b,
           W_ih_l1_f, W_hh_l1_f, b_ih_l1_f, b_hh_l1_f,
           W_ih_l1_b, W_hh_l1_b, b_ih_l1_b, b_hh_l1_b,
           fc_w, fc_b):
    idx = x.T.astype(jnp.int32).reshape(BT)
    e_flat = _sc_gather(emb.astype(jnp.bfloat16), idx)
    e_tbd = e_flat.reshape(T, B, D)

    bf16 = jnp.bfloat16
    gs = _gate_scale()
    wf = (jnp.concatenate([W_ih_l0_f.T, W_hh_l0_f.T], axis=0) * gs).astype(bf16)
    wb = (jnp.concatenate([W_ih_l0_b.T, W_hh_l0_b.T], axis=0) * gs).astype(bf16)
    bf = ((b_ih_l0_f + b_hh_l0_f) * gs).reshape(1, 4 * H)
    bb = ((b_ih_l0_b + b_hh_l0_b) * gs).reshape(1, 4 * H)
    w1f = (jnp.concatenate([W_ih_l1_f.T, W_hh_l1_f.T], axis=0) * gs).astype(bf16)
    b1f = ((b_ih_l1_f + b_hh_l1_f) * gs).reshape(1, 4 * H)
    w1b = (W_ih_l1_b.T * gs).astype(bf16)
    b1b = ((b_ih_l1_b + b_hh_l1_b) * gs).reshape(1, 4 * H)
    fcw = fc_w.T
    fcb = fc_b.reshape(1, 1)
    out = _run_fused(e_tbd, wf, wb, bf, bb, w1f, b1f, w1b, b1b, fcw, fcb)
    return out[:, 0]


# speedup vs baseline: 1.0825x; 1.0825x over previous
import jax
import jax.numpy as jnp
from jax import lax
from jax.experimental import pallas as pl
from jax.experimental.pallas import tpu as pltpu
from jax.experimental.pallas import tpu_sc as plsc

B, T, D, H = 1024, 200, 64, 128
BT = B * T
NC, NS = 2, 16
NW = NC * NS
CHUNK = 128
NCHUNKS = BT // CHUNK
CPW = NCHUNKS // NW
SPG = 8
TB = T // SPG


def _sc_gather_body(table_hbm, idx_hbm, out_hbm, idx_v, rows0, rows1, sem0, sem1):
    wid = lax.axis_index("s") * NC + lax.axis_index("c")
    rbase = wid * CPW
    pltpu.sync_copy(idx_hbm.at[pl.ds(rbase * CHUNK, CPW * CHUNK)], idx_v)

    @pl.loop(0, CPW // 2)
    def _pair(p):
        c0 = p * 2
        d0 = pltpu.async_copy(
            table_hbm.at[idx_v.at[pl.ds(c0 * CHUNK, CHUNK)]], rows0, sem0)
        d1 = pltpu.async_copy(
            table_hbm.at[idx_v.at[pl.ds((c0 + 1) * CHUNK, CHUNK)]], rows1, sem1)
        d0.wait()
        pltpu.sync_copy(rows0, out_hbm.at[pl.ds((rbase + c0) * CHUNK, CHUNK)])
        d1.wait()
        pltpu.sync_copy(rows1, out_hbm.at[pl.ds((rbase + c0 + 1) * CHUNK, CHUNK)])


def _sc_gather(table, idx):
    return pl.kernel(
        _sc_gather_body,
        out_type=jax.ShapeDtypeStruct((BT, D), jnp.bfloat16),
        mesh=plsc.VectorSubcoreMesh(
            core_axis_name="c", subcore_axis_name="s", num_cores=NC, num_subcores=NS
        ),
        scratch_types=[
            pltpu.VMEM((CPW * CHUNK,), jnp.int32),
            pltpu.VMEM((CHUNK, D), jnp.bfloat16),
            pltpu.VMEM((CHUNK, D), jnp.bfloat16),
            pltpu.SemaphoreType.DMA,
            pltpu.SemaphoreType.DMA,
        ],
        compiler_params=pltpu.CompilerParams(use_tc_tiling_on_sc=False),
    )(table, idx)


def _lstm_cell(x_cat, c_prev, w_ref, b_ref):
    g = jnp.dot(x_cat, w_ref[...], preferred_element_type=jnp.float32) + b_ref[...]
    i = 0.5 * jnp.tanh(g[:, 0:H]) + 0.5
    f = 0.5 * jnp.tanh(g[:, H:2 * H]) + 0.5
    gg = jnp.tanh(g[:, 2 * H:3 * H])
    o = 0.5 * jnp.tanh(g[:, 3 * H:4 * H]) + 0.5
    c2 = f * c_prev + i * gg
    h2 = o * jnp.tanh(c2)
    return h2, c2


def _l0_body(ef_ref, eb_ref, wf_ref, wb_ref, bf_ref, bb_ref,
             of_ref, ob_ref, hf, cf, hb, cb):
    t = pl.program_id(0)

    @pl.when(t == 0)
    def _init():
        hf[...] = jnp.zeros_like(hf)
        cf[...] = jnp.zeros_like(cf)
        hb[...] = jnp.zeros_like(hb)
        cb[...] = jnp.zeros_like(cb)

    for s in range(SPG):
        xf = jnp.concatenate([ef_ref[s], hf[...].astype(jnp.bfloat16)], axis=1)
        h2f, c2f = _lstm_cell(xf, cf[...], wf_ref, bf_ref)
        hf[...] = h2f
        cf[...] = c2f
        of_ref[s] = h2f.astype(jnp.bfloat16)

        xb = jnp.concatenate([eb_ref[SPG - 1 - s], hb[...].astype(jnp.bfloat16)], axis=1)
        h2b, c2b = _lstm_cell(xb, cb[...], wb_ref, bb_ref)
        hb[...] = h2b
        cb[...] = c2b
        ob_ref[SPG - 1 - s] = h2b.astype(jnp.bfloat16)


def _run_l0(e_tbd, wf, wb, bf, bb):
    return pl.pallas_call(
        _l0_body,
        grid=(TB,),
        in_specs=[
            pl.BlockSpec((SPG, B, D), lambda t: (t, 0, 0)),
            pl.BlockSpec((SPG, B, D), lambda t: (TB - 1 - t, 0, 0)),
            pl.BlockSpec((D + H, 4 * H), lambda t: (0, 0)),
            pl.BlockSpec((D + H, 4 * H), lambda t: (0, 0)),
            pl.BlockSpec((1, 4 * H), lambda t: (0, 0)),
            pl.BlockSpec((1, 4 * H), lambda t: (0, 0)),
        ],
        out_specs=[
            pl.BlockSpec((SPG, B, H), lambda t: (t, 0, 0)),
            pl.BlockSpec((SPG, B, H), lambda t: (TB - 1 - t, 0, 0)),
        ],
        out_shape=[
            jax.ShapeDtypeStruct((T, B, H), jnp.bfloat16),
            jax.ShapeDtypeStruct((T, B, H), jnp.bfloat16),
        ],
        scratch_shapes=[pltpu.VMEM((B, H), jnp.float32)] * 4,
        compiler_params=pltpu.CompilerParams(dimension_semantics=("arbitrary",)),
    )(e_tbd, e_tbd, wf, wb, bf, bb)


def _l1_body(of_ref, ob_ref, w1f_ref, b1f_ref, w1b_ref, b1b_ref,
             fcw_ref, fcb_ref, out_ref, h1, c1):
    t = pl.program_id(0)

    @pl.when(t == 0)
    def _init():
        h1[...] = jnp.zeros_like(h1)
        c1[...] = jnp.zeros_like(c1)

    cats = []
    h2 = None
    for s in range(SPG):
        cat = jnp.concatenate([of_ref[s], ob_ref[s]], axis=1)
        x = jnp.concatenate([cat, h1[...].astype(jnp.bfloat16)], axis=1)
        h2, c2 = _lstm_cell(x, c1[...], w1f_ref, b1f_ref)
        h1[...] = h2
        c1[...] = c2
        cats.append(cat)

    @pl.when(t == TB - 1)
    def _head():
        cat = cats[SPG - 1]
        gb = jnp.dot(cat, w1b_ref[...], preferred_element_type=jnp.float32) + b1b_ref[...]
        ib = 0.5 * jnp.tanh(gb[:, 0:H]) + 0.5
        ggb = jnp.tanh(gb[:, 2 * H:3 * H])
        ob_g = 0.5 * jnp.tanh(gb[:, 3 * H:4 * H]) + 0.5
        cb2 = ib * ggb
        hb2 = ob_g * jnp.tanh(cb2)
        hcat = jnp.concatenate([h2, hb2], axis=1)
        out_ref[...] = (
            jnp.dot(hcat, fcw_ref[...], preferred_element_type=jnp.float32)
            + fcb_ref[...]
        )


def _run_l1(of, ob, w1f, b1f, w1b, b1b, fcw, fcb):
    return pl.pallas_call(
        _l1_body,
        grid=(TB,),
        in_specs=[
            pl.BlockSpec((SPG, B, H), lambda t: (t, 0, 0)),
            pl.BlockSpec((SPG, B, H), lambda t: (t, 0, 0)),
            pl.BlockSpec((3 * H, 4 * H), lambda t: (0, 0)),
            pl.BlockSpec((1, 4 * H), lambda t: (0, 0)),
            pl.BlockSpec((2 * H, 4 * H), lambda t: (0, 0)),
            pl.BlockSpec((1, 4 * H), lambda t: (0, 0)),
            pl.BlockSpec((2 * H, 1), lambda t: (0, 0)),
            pl.BlockSpec((1, 1), lambda t: (0, 0)),
        ],
        out_specs=pl.BlockSpec((B, 1), lambda t: (0, 0)),
        out_shape=jax.ShapeDtypeStruct((B, 1), jnp.float32),
        scratch_shapes=[pltpu.VMEM((B, H), jnp.float32)] * 2,
        compiler_params=pltpu.CompilerParams(dimension_semantics=("arbitrary",)),
    )(of, ob, w1f, b1f, w1b, b1b, fcw, fcb)


def _gate_scale():
    return jnp.concatenate([
        jnp.full((H,), 0.5, jnp.float32),
        jnp.full((H,), 0.5, jnp.float32),
        jnp.ones((H,), jnp.float32),
        jnp.full((H,), 0.5, jnp.float32),
    ])


def kernel(x, emb,
           W_ih_l0_f, W_hh_l0_f, b_ih_l0_f, b_hh_l0_f,
           W_ih_l0_b, W_hh_l0_b, b_ih_l0_b, b_hh_l0_b,
           W_ih_l1_f, W_hh_l1_f, b_ih_l1_f, b_hh_l1_f,
           W_ih_l1_b, W_hh_l1_b, b_ih_l1_b, b_hh_l1_b,
           fc_w, fc_b):
    idx = x.T.astype(jnp.int32).reshape(BT)
    e_flat = _sc_gather(emb.astype(jnp.bfloat16), idx)
    e_tbd = e_flat.reshape(T, B, D)

    bf16 = jnp.bfloat16
    gs = _gate_scale()
    wf = (jnp.concatenate([W_ih_l0_f.T, W_hh_l0_f.T], axis=0) * gs).astype(bf16)
    wb = (jnp.concatenate([W_ih_l0_b.T, W_hh_l0_b.T], axis=0) * gs).astype(bf16)
    bf = ((b_ih_l0_f + b_hh_l0_f) * gs).reshape(1, 4 * H)
    bb = ((b_ih_l0_b + b_hh_l0_b) * gs).reshape(1, 4 * H)
    of, ob = _run_l0(e_tbd, wf, wb, bf, bb)

    w1f = (jnp.concatenate([W_ih_l1_f.T, W_hh_l1_f.T], axis=0) * gs).astype(bf16)
    b1f = ((b_ih_l1_f + b_hh_l1_f) * gs).reshape(1, 4 * H)
    w1b = (W_ih_l1_b.T * gs).astype(bf16)
    b1b = ((b_ih_l1_b + b_hh_l1_b) * gs).reshape(1, 4 * H)
    fcw = fc_w.T
    fcb = fc_b.reshape(1, 1)
    out = _run_l1(of, ob, w1f, b1f, w1b, b1b, fcw, fcb)
    return out[:, 0]


# speedup vs baseline: 1.0875x; 1.0047x over previous
import jax
import jax.numpy as jnp
from jax import lax
from jax.experimental import pallas as pl
from jax.experimental.pallas import tpu as pltpu
from jax.experimental.pallas import tpu_sc as plsc

B, T, D, H = 1024, 200, 64, 128
BT = B * T
NC, NS = 2, 16
NW = NC * NS
CHUNK = 128
NCHUNKS = BT // CHUNK
CPW = NCHUNKS // NW
SPG = 8
TB = T // SPG


def _sc_gather_body(table_hbm, idx_hbm, out_hbm, idx_v, rows0, rows1, sem0, sem1):
    wid = lax.axis_index("s") * NC + lax.axis_index("c")
    rbase = wid * CPW
    pltpu.sync_copy(idx_hbm.at[pl.ds(rbase * CHUNK, CPW * CHUNK)], idx_v)

    @pl.loop(0, CPW // 2)
    def _pair(p):
        c0 = p * 2
        d0 = pltpu.async_copy(
            table_hbm.at[idx_v.at[pl.ds(c0 * CHUNK, CHUNK)]], rows0, sem0)
        d1 = pltpu.async_copy(
            table_hbm.at[idx_v.at[pl.ds((c0 + 1) * CHUNK, CHUNK)]], rows1, sem1)
        d0.wait()
        pltpu.sync_copy(rows0, out_hbm.at[pl.ds((rbase + c0) * CHUNK, CHUNK)])
        d1.wait()
        pltpu.sync_copy(rows1, out_hbm.at[pl.ds((rbase + c0 + 1) * CHUNK, CHUNK)])


def _sc_gather(table, idx):
    return pl.kernel(
        _sc_gather_body,
        out_type=jax.ShapeDtypeStruct((BT, D), jnp.bfloat16),
        mesh=plsc.VectorSubcoreMesh(
            core_axis_name="c", subcore_axis_name="s", num_cores=NC, num_subcores=NS
        ),
        scratch_types=[
            pltpu.VMEM((CPW * CHUNK,), jnp.int32),
            pltpu.VMEM((CHUNK, D), jnp.bfloat16),
            pltpu.VMEM((CHUNK, D), jnp.bfloat16),
            pltpu.SemaphoreType.DMA,
            pltpu.SemaphoreType.DMA,
        ],
        compiler_params=pltpu.CompilerParams(use_tc_tiling_on_sc=False),
    )(table, idx)


def _lstm_cell(x_cat, c_prev, w_ref, b_ref):
    g = jnp.dot(x_cat, w_ref[...], preferred_element_type=jnp.float32) + b_ref[...]
    i = 0.5 * jnp.tanh(g[:, 0:H]) + 0.5
    f = 0.5 * jnp.tanh(g[:, H:2 * H]) + 0.5
    gg = jnp.tanh(g[:, 2 * H:3 * H])
    o = 0.5 * jnp.tanh(g[:, 3 * H:4 * H]) + 0.5
    c2 = f * c_prev + i * gg
    h2 = o * jnp.tanh(c2)
    return h2, c2


def _l0_body(ef_ref, eb_ref, wf_ref, wb_ref, bf_ref, bb_ref,
             of_ref, ob_ref, hf, cf, hb, cb):
    t = pl.program_id(0)

    @pl.when(t == 0)
    def _init():
        hf[...] = jnp.zeros_like(hf)
        cf[...] = jnp.zeros_like(cf)
        hb[...] = jnp.zeros_like(hb)
        cb[...] = jnp.zeros_like(cb)

    for s in range(SPG):
        xf = jnp.concatenate([ef_ref[s], hf[...].astype(jnp.bfloat16)], axis=1)
        h2f, c2f = _lstm_cell(xf, cf[...], wf_ref, bf_ref)
        hf[...] = h2f
        cf[...] = c2f
        of_ref[s] = h2f.astype(jnp.bfloat16)

        xb = jnp.concatenate([eb_ref[SPG - 1 - s], hb[...].astype(jnp.bfloat16)], axis=1)
        h2b, c2b = _lstm_cell(xb, cb[...], wb_ref, bb_ref)
        hb[...] = h2b
        cb[...] = c2b
        ob_ref[SPG - 1 - s] = h2b.astype(jnp.bfloat16)


def _run_l0(e_tbd, wf, wb, bf, bb):
    return pl.pallas_call(
        _l0_body,
        grid=(TB,),
        in_specs=[
            pl.BlockSpec((SPG, B, D), lambda t: (t, 0, 0)),
            pl.BlockSpec((SPG, B, D), lambda t: (TB - 1 - t, 0, 0)),
            pl.BlockSpec((D + H, 4 * H), lambda t: (0, 0)),
            pl.BlockSpec((D + H, 4 * H), lambda t: (0, 0)),
            pl.BlockSpec((1, 4 * H), lambda t: (0, 0)),
            pl.BlockSpec((1, 4 * H), lambda t: (0, 0)),
        ],
        out_specs=[
            pl.BlockSpec((SPG, B, H), lambda t: (t, 0, 0)),
            pl.BlockSpec((SPG, B, H), lambda t: (TB - 1 - t, 0, 0)),
        ],
        out_shape=[
            jax.ShapeDtypeStruct((T, B, H), jnp.bfloat16),
            jax.ShapeDtypeStruct((T, B, H), jnp.bfloat16),
        ],
        scratch_shapes=[pltpu.VMEM((B, H), jnp.float32)] * 4,
        compiler_params=pltpu.CompilerParams(dimension_semantics=("arbitrary",)),
    )(e_tbd, e_tbd, wf, wb, bf, bb)


def _l1_body(of_ref, ob_ref, w1f_ref, b1f_ref, w1b_ref, b1b_ref,
             fcw_ref, fcb_ref, out_ref, h1, c1):
    t = pl.program_id(0)

    @pl.when(t == 0)
    def _init():
        h1[...] = jnp.zeros_like(h1)
        c1[...] = jnp.zeros_like(c1)

    HALF = B // 2
    last = {}
    for s in range(SPG):
        for k, lo in enumerate((0, HALF)):
            sl = pl.ds(lo, HALF)
            cat = jnp.concatenate([of_ref[s, sl], ob_ref[s, sl]], axis=1)
            x = jnp.concatenate([cat, h1[sl].astype(jnp.bfloat16)], axis=1)
            h2, c2 = _lstm_cell(x, c1[sl], w1f_ref, b1f_ref)
            h1[sl] = h2
            c1[sl] = c2
            last[k] = (cat, h2)

    @pl.when(t == TB - 1)
    def _head():
        for k, lo in enumerate((0, HALF)):
            cat, h2 = last[k]
            gb = jnp.dot(cat, w1b_ref[...], preferred_element_type=jnp.float32) + b1b_ref[...]
            ib = 0.5 * jnp.tanh(gb[:, 0:H]) + 0.5
            ggb = jnp.tanh(gb[:, 2 * H:3 * H])
            ob_g = 0.5 * jnp.tanh(gb[:, 3 * H:4 * H]) + 0.5
            cb2 = ib * ggb
            hb2 = ob_g * jnp.tanh(cb2)
            hcat = jnp.concatenate([h2, hb2], axis=1)
            out_ref[pl.ds(lo, HALF)] = (
                jnp.dot(hcat, fcw_ref[...], preferred_element_type=jnp.float32)
                + fcb_ref[...]
            )


def _run_l1(of, ob, w1f, b1f, w1b, b1b, fcw, fcb):
    return pl.pallas_call(
        _l1_body,
        grid=(TB,),
        in_specs=[
            pl.BlockSpec((SPG, B, H), lambda t: (t, 0, 0)),
            pl.BlockSpec((SPG, B, H), lambda t: (t, 0, 0)),
            pl.BlockSpec((3 * H, 4 * H), lambda t: (0, 0)),
            pl.BlockSpec((1, 4 * H), lambda t: (0, 0)),
            pl.BlockSpec((2 * H, 4 * H), lambda t: (0, 0)),
            pl.BlockSpec((1, 4 * H), lambda t: (0, 0)),
            pl.BlockSpec((2 * H, 1), lambda t: (0, 0)),
            pl.BlockSpec((1, 1), lambda t: (0, 0)),
        ],
        out_specs=pl.BlockSpec((B, 1), lambda t: (0, 0)),
        out_shape=jax.ShapeDtypeStruct((B, 1), jnp.float32),
        scratch_shapes=[pltpu.VMEM((B, H), jnp.float32)] * 2,
        compiler_params=pltpu.CompilerParams(dimension_semantics=("arbitrary",)),
    )(of, ob, w1f, b1f, w1b, b1b, fcw, fcb)


def _gate_scale():
    return jnp.concatenate([
        jnp.full((H,), 0.5, jnp.float32),
        jnp.full((H,), 0.5, jnp.float32),
        jnp.ones((H,), jnp.float32),
        jnp.full((H,), 0.5, jnp.float32),
    ])


def kernel(x, emb,
           W_ih_l0_f, W_hh_l0_f, b_ih_l0_f, b_hh_l0_f,
           W_ih_l0_b, W_hh_l0_b, b_ih_l0_b, b_hh_l0_b,
           W_ih_l1_f, W_hh_l1_f, b_ih_l1_f, b_hh_l1_f,
           W_ih_l1_b, W_hh_l1_b, b_ih_l1_b, b_hh_l1_b,
           fc_w, fc_b):
    idx = x.T.astype(jnp.int32).reshape(BT)
    e_flat = _sc_gather(emb.astype(jnp.bfloat16), idx)
    e_tbd = e_flat.reshape(T, B, D)

    bf16 = jnp.bfloat16
    gs = _gate_scale()
    wf = (jnp.concatenate([W_ih_l0_f.T, W_hh_l0_f.T], axis=0) * gs).astype(bf16)
    wb = (jnp.concatenate([W_ih_l0_b.T, W_hh_l0_b.T], axis=0) * gs).astype(bf16)
    bf = ((b_ih_l0_f + b_hh_l0_f) * gs).reshape(1, 4 * H)
    bb = ((b_ih_l0_b + b_hh_l0_b) * gs).reshape(1, 4 * H)
    of, ob = _run_l0(e_tbd, wf, wb, bf, bb)

    w1f = (jnp.concatenate([W_ih_l1_f.T, W_hh_l1_f.T], axis=0) * gs).astype(bf16)
    b1f = ((b_ih_l1_f + b_hh_l1_f) * gs).reshape(1, 4 * H)
    w1b = (W_ih_l1_b.T * gs).astype(bf16)
    b1b = ((b_ih_l1_b + b_hh_l1_b) * gs).reshape(1, 4 * H)
    fcw = fc_w.T
    fcb = fc_b.reshape(1, 1)
    out = _run_l1(of, ob, w1f, b1f, w1b, b1b, fcw, fcb)
    return out[:, 0]
